# baseline (speedup 1.0000x reference)
import functools

import jax
import jax.numpy as jnp
from jax import lax
from jax.experimental import pallas as pl
from jax.experimental.pallas import tpu as pltpu
from jax.experimental.pallas import tpu_sc as plsc

F32 = jnp.float32
I32 = jnp.int32

N_ITEMS = 20000
N_ENTITIES = 40000
N_NODES = 50000
N_USERS = 10000
D = 128
B = 4096

NC = 2
NS = 16
L = 16

EBLK = 1024
GSUB = 64
RZ = 64
KG_EPAD = 16 * 50 * EBLK
UKG_EPAD = 16 * 10 * EBLK

_SC_PARAMS = pltpu.CompilerParams(needs_layout_passes=False)


def _round_up(x, m):
    return (x + m - 1) // m * m


def _make_deg(N, EPAD):
    CHALF = N // 2
    CHALF_P = _round_up(CHALF, 128)
    EPT = EPAD // NS
    NBLK = EPT // EBLK
    NCH = CHALF_P // 128
    mesh = plsc.VectorSubcoreMesh(core_axis_name="c", subcore_axis_name="s")

    @functools.partial(
        pl.kernel, mesh=mesh, compiler_params=_SC_PARAMS,
        out_type=jax.ShapeDtypeStruct((NC * CHALF_P,), F32),
        scratch_types=[
            pltpu.VMEM((EBLK,), I32),
            pltpu.VMEM((CHALF_P,), F32),
            pltpu.VMEM((NS, 128), F32),
            pltpu.VMEM_SHARED((NS, CHALF_P), F32),
        ],
    )
    def deg(head, out, hblk, cl, red, stag):
        cid = lax.axis_index("c")
        sid = lax.axis_index("s")
        lo = cid * CHALF
        ones = jnp.ones((L,), F32)
        zeros = jnp.zeros((L,), F32)

        def z(r, _):
            cl[pl.ds(r * L, L)] = zeros
            return 0
        lax.fori_loop(0, CHALF_P // L, z, 0)

        def block(b, _):
            b0 = sid * EPT + b * EBLK
            pltpu.sync_copy(head.at[pl.ds(b0, EBLK)], hblk)

            def cb(k, _):
                h = hblk[pl.ds(k * L, L)]
                in_r = (h >= lo) & (h < lo + CHALF)
                plsc.addupdate_scatter(cl, [h - lo], ones, mask=in_r)
                return 0
            lax.fori_loop(0, EBLK // L, cb, 0)
            return 0
        lax.fori_loop(0, NBLK, block, 0)

        pltpu.sync_copy(cl, stag.at[sid])
        plsc.subcore_barrier()

        def wb(i, _):
            r0 = (sid + i * NS) * 128
            pltpu.sync_copy(stag.at[:, pl.ds(r0, 128)], red)
            for j in range(128 // L):
                acc = zeros
                for t in range(NS):
                    acc = acc + red[t, pl.ds(j * L, L)]
                cl[pl.ds(r0 + j * L, L)] = acc
            pltpu.sync_copy(cl.at[pl.ds(r0, 128)],
                            out.at[pl.ds(cid * CHALF_P + r0, 128)])
            return 0
        lax.fori_loop(0, (NCH - sid + NS - 1) // NS, wb, 0)

    return deg


def _make_hop(N, EPAD, NREL, NCHUNK):
    CQ = N // NCHUNK
    EPT = EPAD // NS
    NBLK = EPT // EBLK
    DUMP = CQ
    ACC = _round_up(CQ + 8, NS * RZ)
    ZPT = ACC // (NS * RZ)
    CWB = 4
    NCH = CQ // CWB
    NSUB = EBLK // GSUB
    mesh = plsc.VectorSubcoreMesh(core_axis_name="c", subcore_axis_name="s")

    @functools.partial(
        pl.kernel, mesh=mesh, compiler_params=_SC_PARAMS,
        out_type=jax.ShapeDtypeStruct((N, D), F32),
        scratch_types=[
            pltpu.VMEM((EBLK,), I32),
            pltpu.VMEM((EBLK,), I32),
            pltpu.VMEM((EBLK,), I32),
            pltpu.VMEM((EBLK,), I32),
            pltpu.VMEM((NREL, D), F32),
            pltpu.VMEM((GSUB, D), F32),
            pltpu.VMEM_SHARED((ACC, D), F32),
        ],
    )
    def hop(tab, head, tail, et, rel, zerosD, out,
            hblk, tblk, eblk, ch, rel_l, rows, accum):
        cid = lax.axis_index("c")
        sid = lax.axis_index("s")
        iota = lax.iota(I32, L)
        pltpu.sync_copy(rel, rel_l)

        for q in range(NCHUNK // NC):
            lo = (q * NC + cid) * CQ

            def zb(z, _):
                r0 = (sid * ZPT + z) * RZ
                pltpu.sync_copy(zerosD, accum.at[pl.ds(r0, RZ)])
                return 0
            lax.fori_loop(0, ZPT, zb, 0)
            plsc.subcore_barrier()

            def block(b, _):
                b0 = sid * EPT + b * EBLK
                pltpu.sync_copy(head.at[pl.ds(b0, EBLK)], hblk)
                pltpu.sync_copy(tail.at[pl.ds(b0, EBLK)], tblk)
                pltpu.sync_copy(et.at[pl.ds(b0, EBLK)], eblk)

                def cb(k, _):
                    h = hblk[pl.ds(k * L, L)]
                    in_r = (h >= lo) & (h < lo + CQ)
                    ch[pl.ds(k * L, L)] = jnp.where(in_r, h - lo, DUMP + (h & 7))
                    return 0
                lax.fori_loop(0, EBLK // L, cb, 0)

                def sb(s, _):
                    off = s * GSUB
                    ids = tblk.at[pl.ds(off, GSUB)]
                    pltpu.sync_copy(tab.at[ids], rows)

                    def mg(g, _):
                        eo = off + g * L
                        etv = eblk[pl.ds(eo, L)]
                        rix = iota + g * L
                        for d in range(D):
                            dsp = jnp.full((L,), d, I32)
                            rv = plsc.load_gather(rows, [rix, dsp])
                            relv = plsc.load_gather(rel_l, [etv, dsp])
                            plsc.store_scatter(rows, [rix, dsp], rv * relv)
                        return 0
                    lax.fori_loop(0, GSUB // L, mg, 0)
                    sidx = ch.at[pl.ds(off, GSUB)]
                    pltpu.sync_copy(rows, accum.at[sidx], add=True)
                    return 0
                lax.fori_loop(0, NSUB, sb, 0)
                return 0
            lax.fori_loop(0, NBLK, block, 0)
            plsc.subcore_barrier()

            def wb(i, _):
                r0 = (sid + i * NS) * CWB
                pltpu.sync_copy(accum.at[pl.ds(r0, CWB)],
                                out.at[pl.ds(lo + r0, CWB)])
                return 0
            lax.fori_loop(0, (NCH - sid + NS - 1) // NS, wb, 0)
            plsc.subcore_barrier()

    return hop


def _make_gather():
    RPW = B // (NC * NS)
    mesh = plsc.VectorSubcoreMesh(core_axis_name="c", subcore_axis_name="s")

    @functools.partial(
        pl.kernel, mesh=mesh, compiler_params=_SC_PARAMS,
        out_type=jax.ShapeDtypeStruct((3, B, D), F32),
        scratch_types=[
            pltpu.VMEM((RPW,), I32),
            pltpu.VMEM((RPW,), I32),
            pltpu.VMEM((RPW,), I32),
            pltpu.VMEM((RPW, D), F32),
        ],
    )
    def gather(allT, fuT, uids, iids, out, uv, ulv, iv, rows):
        cid = lax.axis_index("c")
        sid = lax.axis_index("s")
        wid = sid * NC + cid
        base = wid * RPW
        pltpu.sync_copy(uids.at[pl.ds(base, RPW)], uv)
        pltpu.sync_copy(iids.at[pl.ds(base, RPW)], iv)

        def adj(k, _):
            ulv[pl.ds(k * L, L)] = uv[pl.ds(k * L, L)] - N_ENTITIES
            return 0
        lax.fori_loop(0, RPW // L, adj, 0)

        for j, (tab, idx) in enumerate(((fuT, ulv), (allT, uv), (allT, iv))):
            pltpu.sync_copy(tab.at[idx], rows)
            pltpu.sync_copy(rows, out.at[j].at[pl.ds(base, RPW)])

    return gather


def _mlp(image_features, text_features, W_img1, b_img1, W_img2, b_img2,
         W_txt1, b_txt1, W_txt2, b_txt2):
    n = image_features.shape[0]
    blk = 1000
    grid = n // blk

    def body(xi_ref, xt_ref, wi1, bi1, wi2, bi2, wt1, bt1, wt2, bt2, o_ref):
        dn = (((1,), (1,)), ((), ()))
        hi = lax.dot_general(xi_ref[...], wi1[...], dn,
                             preferred_element_type=F32) + bi1[...]
        hi = jnp.where(hi > 0, hi, 0.01 * hi)
        oi = lax.dot_general(hi, wi2[...], dn,
                             preferred_element_type=F32) + bi2[...]
        ht = lax.dot_general(xt_ref[...], wt1[...], dn,
                             preferred_element_type=F32) + bt1[...]
        ht = jnp.where(ht > 0, ht, 0.01 * ht)
        ot = lax.dot_general(ht, wt2[...], dn,
                             preferred_element_type=F32) + bt2[...]
        o_ref[...] = jnp.concatenate([oi, ot], axis=1)

    full = lambda a: pl.BlockSpec(a.shape, lambda i: (0,) * a.ndim)
    return pl.pallas_call(
        body,
        grid=(grid,),
        in_specs=[
            pl.BlockSpec((blk, image_features.shape[1]), lambda i: (i, 0)),
            pl.BlockSpec((blk, text_features.shape[1]), lambda i: (i, 0)),
            full(W_img1), full(b_img1), full(W_img2), full(b_img2),
            full(W_txt1), full(b_txt1), full(W_txt2), full(b_txt2),
        ],
        out_specs=pl.BlockSpec((blk, D), lambda i: (i, 0)),
        out_shape=jax.ShapeDtypeStruct((n, D), F32),
    )(image_features, text_features, W_img1, b_img1, W_img2, b_img2,
      W_txt1, b_txt1, W_txt2, b_txt2)


def _scale(raw, counts):
    n = raw.shape[0]
    blk = 1000
    grid = n // blk

    def body(r, c, o):
        inv = 1.0 / jnp.maximum(c[:, :1], 1.0)
        o[...] = r[...] * inv

    bs = pl.BlockSpec((blk, D), lambda i: (i, 0))
    cs = pl.BlockSpec((blk, 16), lambda i: (i, 0))
    return pl.pallas_call(
        body, grid=(grid,), in_specs=[bs, cs], out_specs=bs,
        out_shape=jax.ShapeDtypeStruct((n, D), F32),
    )(raw, counts)


def _scale_combine(raw, counts, ego, agg1):
    n = raw.shape[0]
    blk = 1000
    grid = n // blk

    def body(r, c, e, a, o):
        inv = 1.0 / jnp.maximum(c[:, :1], 1.0)
        o[...] = (e[...] + a[...] + r[...] * inv) * (1.0 / 3.0)

    bs = pl.BlockSpec((blk, D), lambda i: (i, 0))
    cs = pl.BlockSpec((blk, 16), lambda i: (i, 0))
    return pl.pallas_call(
        body, grid=(grid,), in_specs=[bs, cs, bs, bs], out_specs=bs,
        out_shape=jax.ShapeDtypeStruct((n, D), F32),
    )(raw, counts, ego, agg1)


def _gating(gath, g1W, g1b, g2W, g2b, g3W, g3b, g4W, g4b):
    blk = 512
    grid = B // blk

    def body(g, w1, c1, w2, c2, w3, c3, w4, c4, o):
        dn = (((1,), (1,)), ((), ()))
        a = g[0]
        bb = g[1]
        it = g[2]
        a_i, a_t = a[:, :64], a[:, 64:]
        b_i, b_t = bb[:, :64], bb[:, 64:]
        z1 = (lax.dot_general(a_i, w1[...], dn, preferred_element_type=F32)
              + c1[...]
              + lax.dot_general(b_i, w2[...], dn, preferred_element_type=F32)
              + c2[...])
        gi1 = 1.0 / (1.0 + jnp.exp(-z1))
        uf_i = gi1 * a_i + (1.0 - gi1) * b_i
        z2 = (lax.dot_general(a_t, w3[...], dn, preferred_element_type=F32)
              + c3[...]
              + lax.dot_general(b_t, w4[...], dn, preferred_element_type=F32)
              + c4[...])
        gi2 = 1.0 / (1.0 + jnp.exp(-z2))
        uf_t = gi2 * a_t + (1.0 - gi2) * b_t
        uf = jnp.concatenate([uf_i, uf_t], axis=1)
        s = jnp.sum(uf * it, axis=1)
        o[0, :] = 1.0 / (1.0 + jnp.exp(-s))

    full = lambda a: pl.BlockSpec(a.shape, lambda i: (0,) * a.ndim)
    out = pl.pallas_call(
        body, grid=(grid,),
        in_specs=[pl.BlockSpec((3, blk, D), lambda i: (0, i, 0)),
                  full(g1W), full(g1b), full(g2W), full(g2b),
                  full(g3W), full(g3b), full(g4W), full(g4b)],
        out_specs=pl.BlockSpec((1, blk), lambda i: (0, i)),
        out_shape=jax.ShapeDtypeStruct((1, B), F32),
    )(gath, g1W, g1b, g2W, g2b, g3W, g3b, g4W, g4b)
    return out.reshape(B)


_deg_kg = _make_deg(N_NODES, KG_EPAD)
_deg_ukg = _make_deg(N_USERS, UKG_EPAD)
_hop_kg = _make_hop(N_NODES, KG_EPAD, 32, 4)
_hop_ukg = _make_hop(N_USERS, UKG_EPAD, 8, 2)
_gather3 = _make_gather()


def _pad_edges(head, tail, et, epad, n):
    e = head.shape[0]
    pad = epad - e
    headp = jnp.concatenate([head, jnp.full((pad,), n, I32)])
    tailp = jnp.concatenate([tail, jnp.zeros((pad,), I32)])
    etp = jnp.concatenate([et, jnp.zeros((pad,), I32)])
    return headp, tailp, etp


_STAGE = 15


def _seg_raw(tab, head, tail, et, rel, n):
    return jax.ops.segment_sum(tab[tail] * rel[et], head, num_segments=n)


def kernel(user_ids, item_ids, edge_index, edge_type, ukg_edge_index,
           ukg_edge_type, image_features, text_features,
           W_img1, b_img1, W_img2, b_img2, W_txt1, b_txt1, W_txt2, b_txt2,
           other_emb_image, other_emb_text, rel_emb_image, rel_emb_text,
           ukg_rel_emb_image, ukg_rel_emb_text,
           g1W, g1b, g2W, g2b, g3W, g3b, g4W, g4b):
    zerosD = jnp.zeros((RZ, D), F32)
    zeros16 = jnp.zeros((RZ, 16), F32)
    ones16 = jnp.ones((GSUB, 16), F32)

    mlp_out = _mlp(image_features, text_features, W_img1, b_img1, W_img2,
                   b_img2, W_txt1, b_txt1, W_txt2, b_txt2)
    other = jnp.concatenate([other_emb_image, other_emb_text], axis=1)
    ego = jnp.concatenate([mlp_out, other], axis=0)
    rel = jnp.concatenate([rel_emb_image, rel_emb_text], axis=1)
    urel = jnp.concatenate([ukg_rel_emb_image, ukg_rel_emb_text], axis=1)

    head, tail = edge_index[0], edge_index[1]
    uhead, utail = ukg_edge_index[0], ukg_edge_index[1]
    headp, tailp, etp = _pad_edges(head, tail, edge_type, KG_EPAD, N_NODES)
    uheadp, utailp, uetp = _pad_edges(uhead, utail, ukg_edge_type,
                                      UKG_EPAD, N_USERS)

    if _STAGE & 1:
        cpad = _deg_kg(headp)
        hp = _round_up(N_NODES // 2, 128)
        counts1 = jnp.concatenate([cpad[:N_NODES // 2],
                                   cpad[hp:hp + N_NODES // 2]])
        counts = jnp.broadcast_to(counts1[:, None], (N_NODES, 16))
        ucpad = _deg_ukg(uheadp)
        uhp = _round_up(N_USERS // 2, 128)
        ucounts1 = jnp.concatenate([ucpad[:N_USERS // 2],
                                    ucpad[uhp:uhp + N_USERS // 2]])
        ucounts = jnp.broadcast_to(ucounts1[:, None], (N_USERS, 16))
    else:
        counts = jnp.broadcast_to(jax.ops.segment_sum(
            jnp.ones(head.shape[0], F32), head,
            num_segments=N_NODES)[:, None], (N_NODES, 16))
        ucounts = jnp.broadcast_to(jax.ops.segment_sum(
            jnp.ones(uhead.shape[0], F32), uhead,
            num_segments=N_USERS)[:, None], (N_USERS, 16))

    if _STAGE & 2:
        raw1 = _hop_kg(ego, headp, tailp, etp, rel, zerosD)
        agg1 = _scale(raw1, counts)
        raw2 = _hop_kg(agg1, headp, tailp, etp, rel, zerosD)
    else:
        raw1 = _seg_raw(ego, head, tail, edge_type, rel, N_NODES)
        agg1 = _scale(raw1, counts)
        raw2 = _seg_raw(agg1, head, tail, edge_type, rel, N_NODES)
    allT = _scale_combine(raw2, counts, ego, agg1)

    uego = allT[N_ENTITIES:]
    if _STAGE & 4:
        uraw1 = _hop_ukg(uego, uheadp, utailp, uetp, urel, zerosD)
        uagg1 = _scale(uraw1, ucounts)
        uraw2 = _hop_ukg(uagg1, uheadp, utailp, uetp, urel, zerosD)
    else:
        uraw1 = _seg_raw(uego, uhead, utail, ukg_edge_type, urel, N_USERS)
        uagg1 = _scale(uraw1, ucounts)
        uraw2 = _seg_raw(uagg1, uhead, utail, ukg_edge_type, urel, N_USERS)
    fuT = _scale_combine(uraw2, ucounts, uego, uagg1)

    if _STAGE & 8:
        gath = _gather3(allT, fuT, user_ids, item_ids)
    else:
        ul = user_ids - N_ENTITIES
        gath = jnp.stack([fuT[ul], allT[user_ids], allT[item_ids]])
    return _gating(gath, g1W, g1b, g2W, g2b, g3W, g3b, g4W, g4b)


# speedup vs baseline: 3.0866x; 3.0866x over previous
import functools

import jax
import jax.numpy as jnp
from jax import lax
from jax.experimental import pallas as pl
from jax.experimental.pallas import tpu as pltpu
from jax.experimental.pallas import tpu_sc as plsc

F32 = jnp.float32
I32 = jnp.int32

N_ITEMS = 20000
N_ENTITIES = 40000
N_NODES = 50000
N_USERS = 10000
D = 128
B = 4096

NC = 2
NS = 16
L = 16

EBLK = 1024
GSUB = 64
RZ = 64
KG_EPAD = 16 * 50 * EBLK
UKG_EPAD = 16 * 10 * EBLK

_SC_PARAMS = pltpu.CompilerParams(needs_layout_passes=False)


def _round_up(x, m):
    return (x + m - 1) // m * m


def _make_deg(N, EPAD):
    CHALF = N // 2
    CHALF_P = _round_up(CHALF, 128)
    EPT = EPAD // NS
    NBLK = EPT // EBLK
    NCH = CHALF_P // 128
    mesh = plsc.VectorSubcoreMesh(core_axis_name="c", subcore_axis_name="s")

    @functools.partial(
        pl.kernel, mesh=mesh, compiler_params=_SC_PARAMS,
        out_type=jax.ShapeDtypeStruct((NC * CHALF_P,), F32),
        scratch_types=[
            pltpu.VMEM((EBLK,), I32),
            pltpu.VMEM((CHALF_P,), F32),
            pltpu.VMEM((NS, 128), F32),
            pltpu.VMEM_SHARED((NS, CHALF_P), F32),
        ],
    )
    def deg(head, out, hblk, cl, red, stag):
        cid = lax.axis_index("c")
        sid = lax.axis_index("s")
        lo = cid * CHALF
        ones = jnp.ones((L,), F32)
        zeros = jnp.zeros((L,), F32)

        def z(r, _):
            cl[pl.ds(r * L, L)] = zeros
            return 0
        lax.fori_loop(0, CHALF_P // L, z, 0)

        def block(b, _):
            b0 = sid * EPT + b * EBLK
            pltpu.sync_copy(head.at[pl.ds(b0, EBLK)], hblk)

            def cb(k, _):
                h = hblk[pl.ds(k * L, L)]
                in_r = (h >= lo) & (h < lo + CHALF)
                plsc.addupdate_scatter(cl, [h - lo], ones, mask=in_r)
                return 0
            lax.fori_loop(0, EBLK // L, cb, 0)
            return 0
        lax.fori_loop(0, NBLK, block, 0)

        pltpu.sync_copy(cl, stag.at[sid])
        plsc.subcore_barrier()

        def wb(i, _):
            r0 = (sid + i * NS) * 128
            pltpu.sync_copy(stag.at[:, pl.ds(r0, 128)], red)
            for j in range(128 // L):
                acc = zeros
                for t in range(NS):
                    acc = acc + red[t, pl.ds(j * L, L)]
                cl[pl.ds(r0 + j * L, L)] = acc
            pltpu.sync_copy(cl.at[pl.ds(r0, 128)],
                            out.at[pl.ds(cid * CHALF_P + r0, 128)])
            return 0
        lax.fori_loop(0, (NCH - sid + NS - 1) // NS, wb, 0)

    return deg


def _make_hop(N, EPAD, NREL, NCHUNK):
    CQ = N // NCHUNK
    EPT = EPAD // NS
    NBLK = EPT // EBLK
    DUMP = CQ
    ACC = _round_up(CQ + 64, NS * RZ)
    ZPT = ACC // (NS * RZ)
    CWB = 4
    NCH = CQ // CWB
    NSUB = EBLK // GSUB
    mesh = plsc.VectorSubcoreMesh(core_axis_name="c", subcore_axis_name="s")

    @functools.partial(
        pl.kernel, mesh=mesh, compiler_params=_SC_PARAMS,
        out_type=jax.ShapeDtypeStruct((N, D), F32),
        scratch_types=[
            pltpu.VMEM((EBLK,), I32),
            pltpu.VMEM((EBLK,), I32),
            pltpu.VMEM((EBLK,), I32),
            pltpu.VMEM((EBLK,), I32),
            pltpu.VMEM((NREL, D), F32),
            pltpu.VMEM((GSUB, D), F32),
            pltpu.VMEM_SHARED((ACC, D), F32),
        ],
    )
    def hop(tab, head, tail, et, rel, zerosD, out,
            hblk, tblk, eblk, ch, rel_l, rows, accum):
        cid = lax.axis_index("c")
        sid = lax.axis_index("s")
        iota = lax.iota(I32, L)
        pltpu.sync_copy(rel, rel_l)

        for q in range(NCHUNK // NC):
            lo = (q * NC + cid) * CQ

            def zb(z, _):
                r0 = (sid * ZPT + z) * RZ
                pltpu.sync_copy(zerosD, accum.at[pl.ds(r0, RZ)])
                return 0
            lax.fori_loop(0, ZPT, zb, 0)
            plsc.subcore_barrier()

            def block(b, _):
                b0 = sid * EPT + b * EBLK
                pltpu.sync_copy(head.at[pl.ds(b0, EBLK)], hblk)
                pltpu.sync_copy(tail.at[pl.ds(b0, EBLK)], tblk)
                pltpu.sync_copy(et.at[pl.ds(b0, EBLK)], eblk)

                def cb(k, _):
                    h = hblk[pl.ds(k * L, L)]
                    in_r = (h >= lo) & (h < lo + CQ)
                    ch[pl.ds(k * L, L)] = jnp.where(in_r, h - lo, DUMP + (h & 63))
                    return 0
                lax.fori_loop(0, EBLK // L, cb, 0)

                def sb(s, _):
                    off = s * GSUB
                    ids = tblk.at[pl.ds(off, GSUB)]
                    pltpu.sync_copy(tab.at[ids], rows)

                    def mg(g, _):
                        eo = off + g * L
                        etv = eblk[pl.ds(eo, L)]
                        for l in range(L):
                            ei = etv[l]
                            ri = g * L + l
                            for j in range(D // L):
                                rv = rows[ri, pl.ds(j * L, L)]
                                relv = rel_l[ei, pl.ds(j * L, L)]
                                rows[ri, pl.ds(j * L, L)] = rv * relv
                        return 0
                    lax.fori_loop(0, GSUB // L, mg, 0)
                    sidx = ch.at[pl.ds(off, GSUB)]
                    pltpu.sync_copy(rows, accum.at[sidx], add=True)
                    return 0
                lax.fori_loop(0, NSUB, sb, 0)
                return 0
            lax.fori_loop(0, NBLK, block, 0)
            plsc.subcore_barrier()

            def wb(i, _):
                r0 = (sid + i * NS) * CWB
                pltpu.sync_copy(accum.at[pl.ds(r0, CWB)],
                                out.at[pl.ds(lo + r0, CWB)])
                return 0
            lax.fori_loop(0, (NCH - sid + NS - 1) // NS, wb, 0)
            plsc.subcore_barrier()

    return hop


def _make_gather():
    RPW = B // (NC * NS)
    mesh = plsc.VectorSubcoreMesh(core_axis_name="c", subcore_axis_name="s")

    @functools.partial(
        pl.kernel, mesh=mesh, compiler_params=_SC_PARAMS,
        out_type=jax.ShapeDtypeStruct((3, B, D), F32),
        scratch_types=[
            pltpu.VMEM((RPW,), I32),
            pltpu.VMEM((RPW,), I32),
            pltpu.VMEM((RPW,), I32),
            pltpu.VMEM((RPW, D), F32),
        ],
    )
    def gather(allT, fuT, uids, iids, out, uv, ulv, iv, rows):
        cid = lax.axis_index("c")
        sid = lax.axis_index("s")
        wid = sid * NC + cid
        base = wid * RPW
        pltpu.sync_copy(uids.at[pl.ds(base, RPW)], uv)
        pltpu.sync_copy(iids.at[pl.ds(base, RPW)], iv)

        def adj(k, _):
            ulv[pl.ds(k * L, L)] = uv[pl.ds(k * L, L)] - N_ENTITIES
            return 0
        lax.fori_loop(0, RPW // L, adj, 0)

        for j, (tab, idx) in enumerate(((fuT, ulv), (allT, uv), (allT, iv))):
            pltpu.sync_copy(tab.at[idx], rows)
            pltpu.sync_copy(rows, out.at[j].at[pl.ds(base, RPW)])

    return gather


def _mlp(image_features, text_features, W_img1, b_img1, W_img2, b_img2,
         W_txt1, b_txt1, W_txt2, b_txt2):
    n = image_features.shape[0]
    blk = 1000
    grid = n // blk

    def body(xi_ref, xt_ref, wi1, bi1, wi2, bi2, wt1, bt1, wt2, bt2, o_ref):
        dn = (((1,), (1,)), ((), ()))
        hi = lax.dot_general(xi_ref[...], wi1[...], dn,
                             preferred_element_type=F32) + bi1[...]
        hi = jnp.where(hi > 0, hi, 0.01 * hi)
        oi = lax.dot_general(hi, wi2[...], dn,
                             preferred_element_type=F32) + bi2[...]
        ht = lax.dot_general(xt_ref[...], wt1[...], dn,
                             preferred_element_type=F32) + bt1[...]
        ht = jnp.where(ht > 0, ht, 0.01 * ht)
        ot = lax.dot_general(ht, wt2[...], dn,
                             preferred_element_type=F32) + bt2[...]
        o_ref[...] = jnp.concatenate([oi, ot], axis=1)

    full = lambda a: pl.BlockSpec(a.shape, lambda i: (0,) * a.ndim)
    return pl.pallas_call(
        body,
        grid=(grid,),
        in_specs=[
            pl.BlockSpec((blk, image_features.shape[1]), lambda i: (i, 0)),
            pl.BlockSpec((blk, text_features.shape[1]), lambda i: (i, 0)),
            full(W_img1), full(b_img1), full(W_img2), full(b_img2),
            full(W_txt1), full(b_txt1), full(W_txt2), full(b_txt2),
        ],
        out_specs=pl.BlockSpec((blk, D), lambda i: (i, 0)),
        out_shape=jax.ShapeDtypeStruct((n, D), F32),
    )(image_features, text_features, W_img1, b_img1, W_img2, b_img2,
      W_txt1, b_txt1, W_txt2, b_txt2)


def _scale(raw, counts):
    n = raw.shape[0]
    blk = 1000
    grid = n // blk

    def body(r, c, o):
        inv = 1.0 / jnp.maximum(c[:, :1], 1.0)
        o[...] = r[...] * inv

    bs = pl.BlockSpec((blk, D), lambda i: (i, 0))
    cs = pl.BlockSpec((blk, 16), lambda i: (i, 0))
    return pl.pallas_call(
        body, grid=(grid,), in_specs=[bs, cs], out_specs=bs,
        out_shape=jax.ShapeDtypeStruct((n, D), F32),
    )(raw, counts)


def _scale_combine(raw, counts, ego, agg1):
    n = raw.shape[0]
    blk = 1000
    grid = n // blk

    def body(r, c, e, a, o):
        inv = 1.0 / jnp.maximum(c[:, :1], 1.0)
        o[...] = (e[...] + a[...] + r[...] * inv) * (1.0 / 3.0)

    bs = pl.BlockSpec((blk, D), lambda i: (i, 0))
    cs = pl.BlockSpec((blk, 16), lambda i: (i, 0))
    return pl.pallas_call(
        body, grid=(grid,), in_specs=[bs, cs, bs, bs], out_specs=bs,
        out_shape=jax.ShapeDtypeStruct((n, D), F32),
    )(raw, counts, ego, agg1)


def _gating(gath, g1W, g1b, g2W, g2b, g3W, g3b, g4W, g4b):
    blk = 512
    grid = B // blk

    def body(g, w1, c1, w2, c2, w3, c3, w4, c4, o):
        dn = (((1,), (1,)), ((), ()))
        a = g[0]
        bb = g[1]
        it = g[2]
        a_i, a_t = a[:, :64], a[:, 64:]
        b_i, b_t = bb[:, :64], bb[:, 64:]
        z1 = (lax.dot_general(a_i, w1[...], dn, preferred_element_type=F32)
              + c1[...]
              + lax.dot_general(b_i, w2[...], dn, preferred_element_type=F32)
              + c2[...])
        gi1 = 1.0 / (1.0 + jnp.exp(-z1))
        uf_i = gi1 * a_i + (1.0 - gi1) * b_i
        z2 = (lax.dot_general(a_t, w3[...], dn, preferred_element_type=F32)
              + c3[...]
              + lax.dot_general(b_t, w4[...], dn, preferred_element_type=F32)
              + c4[...])
        gi2 = 1.0 / (1.0 + jnp.exp(-z2))
        uf_t = gi2 * a_t + (1.0 - gi2) * b_t
        uf = jnp.concatenate([uf_i, uf_t], axis=1)
        s = jnp.sum(uf * it, axis=1)
        o[0, :] = 1.0 / (1.0 + jnp.exp(-s))

    full = lambda a: pl.BlockSpec(a.shape, lambda i: (0,) * a.ndim)
    out = pl.pallas_call(
        body, grid=(grid,),
        in_specs=[pl.BlockSpec((3, blk, D), lambda i: (0, i, 0)),
                  full(g1W), full(g1b), full(g2W), full(g2b),
                  full(g3W), full(g3b), full(g4W), full(g4b)],
        out_specs=pl.BlockSpec((1, blk), lambda i: (0, i)),
        out_shape=jax.ShapeDtypeStruct((1, B), F32),
    )(gath, g1W, g1b, g2W, g2b, g3W, g3b, g4W, g4b)
    return out.reshape(B)


_deg_kg = _make_deg(N_NODES, KG_EPAD)
_deg_ukg = _make_deg(N_USERS, UKG_EPAD)
_hop_kg = _make_hop(N_NODES, KG_EPAD, 32, 4)
_hop_ukg = _make_hop(N_USERS, UKG_EPAD, 8, 2)
_gather3 = _make_gather()


def _pad_edges(head, tail, et, epad, n):
    e = head.shape[0]
    pad = epad - e
    headp = jnp.concatenate([head, jnp.full((pad,), n, I32)])
    tailp = jnp.concatenate([tail, jnp.zeros((pad,), I32)])
    etp = jnp.concatenate([et, jnp.zeros((pad,), I32)])
    return headp, tailp, etp


_STAGE = 15


def _seg_raw(tab, head, tail, et, rel, n):
    return jax.ops.segment_sum(tab[tail] * rel[et], head, num_segments=n)


def kernel(user_ids, item_ids, edge_index, edge_type, ukg_edge_index,
           ukg_edge_type, image_features, text_features,
           W_img1, b_img1, W_img2, b_img2, W_txt1, b_txt1, W_txt2, b_txt2,
           other_emb_image, other_emb_text, rel_emb_image, rel_emb_text,
           ukg_rel_emb_image, ukg_rel_emb_text,
           g1W, g1b, g2W, g2b, g3W, g3b, g4W, g4b):
    zerosD = jnp.zeros((RZ, D), F32)
    zeros16 = jnp.zeros((RZ, 16), F32)
    ones16 = jnp.ones((GSUB, 16), F32)

    mlp_out = _mlp(image_features, text_features, W_img1, b_img1, W_img2,
                   b_img2, W_txt1, b_txt1, W_txt2, b_txt2)
    other = jnp.concatenate([other_emb_image, other_emb_text], axis=1)
    ego = jnp.concatenate([mlp_out, other], axis=0)
    rel = jnp.concatenate([rel_emb_image, rel_emb_text], axis=1)
    urel = jnp.concatenate([ukg_rel_emb_image, ukg_rel_emb_text], axis=1)

    head, tail = edge_index[0], edge_index[1]
    uhead, utail = ukg_edge_index[0], ukg_edge_index[1]
    headp, tailp, etp = _pad_edges(head, tail, edge_type, KG_EPAD, N_NODES)
    uheadp, utailp, uetp = _pad_edges(uhead, utail, ukg_edge_type,
                                      UKG_EPAD, N_USERS)

    if _STAGE & 1:
        cpad = _deg_kg(headp)
        hp = _round_up(N_NODES // 2, 128)
        counts1 = jnp.concatenate([cpad[:N_NODES // 2],
                                   cpad[hp:hp + N_NODES // 2]])
        counts = jnp.broadcast_to(counts1[:, None], (N_NODES, 16))
        ucpad = _deg_ukg(uheadp)
        uhp = _round_up(N_USERS // 2, 128)
        ucounts1 = jnp.concatenate([ucpad[:N_USERS // 2],
                                    ucpad[uhp:uhp + N_USERS // 2]])
        ucounts = jnp.broadcast_to(ucounts1[:, None], (N_USERS, 16))
    else:
        counts = jnp.broadcast_to(jax.ops.segment_sum(
            jnp.ones(head.shape[0], F32), head,
            num_segments=N_NODES)[:, None], (N_NODES, 16))
        ucounts = jnp.broadcast_to(jax.ops.segment_sum(
            jnp.ones(uhead.shape[0], F32), uhead,
            num_segments=N_USERS)[:, None], (N_USERS, 16))

    if _STAGE & 2:
        raw1 = _hop_kg(ego, headp, tailp, etp, rel, zerosD)
        agg1 = _scale(raw1, counts)
        raw2 = _hop_kg(agg1, headp, tailp, etp, rel, zerosD)
    else:
        raw1 = _seg_raw(ego, head, tail, edge_type, rel, N_NODES)
        agg1 = _scale(raw1, counts)
        raw2 = _seg_raw(agg1, head, tail, edge_type, rel, N_NODES)
    allT = _scale_combine(raw2, counts, ego, agg1)

    uego = allT[N_ENTITIES:]
    if _STAGE & 4:
        uraw1 = _hop_ukg(uego, uheadp, utailp, uetp, urel, zerosD)
        uagg1 = _scale(uraw1, ucounts)
        uraw2 = _hop_ukg(uagg1, uheadp, utailp, uetp, urel, zerosD)
    else:
        uraw1 = _seg_raw(uego, uhead, utail, ukg_edge_type, urel, N_USERS)
        uagg1 = _scale(uraw1, ucounts)
        uraw2 = _seg_raw(uagg1, uhead, utail, ukg_edge_type, urel, N_USERS)
    fuT = _scale_combine(uraw2, ucounts, uego, uagg1)

    if _STAGE & 8:
        gath = _gather3(allT, fuT, user_ids, item_ids)
    else:
        ul = user_ids - N_ENTITIES
        gath = jnp.stack([fuT[ul], allT[user_ids], allT[item_ids]])
    return _gating(gath, g1W, g1b, g2W, g2b, g3W, g3b, g4W, g4b)


# speedup vs baseline: 3.7604x; 1.2183x over previous
import functools

import jax
import jax.numpy as jnp
from jax import lax
from jax.experimental import pallas as pl
from jax.experimental.pallas import tpu as pltpu
from jax.experimental.pallas import tpu_sc as plsc

F32 = jnp.float32
I32 = jnp.int32

N_ITEMS = 20000
N_ENTITIES = 40000
N_NODES = 50000
N_USERS = 10000
D = 128
B = 4096

NC = 2
NS = 16
L = 16

EBLK = 1024
GSUB = 64
RZ = 64
KG_EPAD = 16 * 50 * EBLK
UKG_EPAD = 16 * 10 * EBLK

_SC_PARAMS = pltpu.CompilerParams(needs_layout_passes=False)


def _round_up(x, m):
    return (x + m - 1) // m * m


def _make_deg(N, EPAD):
    CHALF = N // 2
    CHALF_P = _round_up(CHALF, 128)
    EPT = EPAD // NS
    NBLK = EPT // EBLK
    NCH = CHALF_P // 128
    mesh = plsc.VectorSubcoreMesh(core_axis_name="c", subcore_axis_name="s")

    @functools.partial(
        pl.kernel, mesh=mesh, compiler_params=_SC_PARAMS,
        out_type=jax.ShapeDtypeStruct((NC * CHALF_P,), F32),
        scratch_types=[
            pltpu.VMEM((EBLK,), I32),
            pltpu.VMEM((CHALF_P,), F32),
            pltpu.VMEM((NS, 128), F32),
            pltpu.VMEM_SHARED((NS, CHALF_P), F32),
        ],
    )
    def deg(head, out, hblk, cl, red, stag):
        cid = lax.axis_index("c")
        sid = lax.axis_index("s")
        lo = cid * CHALF
        ones = jnp.ones((L,), F32)
        zeros = jnp.zeros((L,), F32)

        def z(r, _):
            cl[pl.ds(r * L, L)] = zeros
            return 0
        lax.fori_loop(0, CHALF_P // L, z, 0)

        def block(b, _):
            b0 = sid * EPT + b * EBLK
            pltpu.sync_copy(head.at[pl.ds(b0, EBLK)], hblk)

            def cb(k, _):
                h = hblk[pl.ds(k * L, L)]
                in_r = (h >= lo) & (h < lo + CHALF)
                plsc.addupdate_scatter(cl, [h - lo], ones, mask=in_r)
                return 0
            lax.fori_loop(0, EBLK // L, cb, 0)
            return 0
        lax.fori_loop(0, NBLK, block, 0)

        pltpu.sync_copy(cl, stag.at[sid])
        plsc.subcore_barrier()

        def wb(i, _):
            r0 = (sid + i * NS) * 128
            pltpu.sync_copy(stag.at[:, pl.ds(r0, 128)], red)
            for j in range(128 // L):
                acc = zeros
                for t in range(NS):
                    acc = acc + red[t, pl.ds(j * L, L)]
                cl[pl.ds(r0 + j * L, L)] = acc
            pltpu.sync_copy(cl.at[pl.ds(r0, 128)],
                            out.at[pl.ds(cid * CHALF_P + r0, 128)])
            return 0
        lax.fori_loop(0, (NCH - sid + NS - 1) // NS, wb, 0)

    return deg


def _make_hop(N, EPAD, NREL, NCHUNK):
    CQ = N // NCHUNK
    EPT = EPAD // NS
    NBLK = EPT // EBLK
    DUMP = CQ
    ACC = _round_up(CQ + 64, NS * RZ)
    ZPT = ACC // (NS * RZ)
    CWB = 4
    NCH = CQ // CWB
    NSUB = EBLK // GSUB
    mesh = plsc.VectorSubcoreMesh(core_axis_name="c", subcore_axis_name="s")

    @functools.partial(
        pl.kernel, mesh=mesh, compiler_params=_SC_PARAMS,
        out_type=jax.ShapeDtypeStruct((N, D), F32),
        scratch_types=[
            pltpu.VMEM((EBLK,), I32),
            pltpu.VMEM((EBLK,), I32),
            pltpu.VMEM((EBLK,), I32),
            pltpu.VMEM((EBLK,), I32),
            pltpu.VMEM((NREL, D), F32),
            pltpu.VMEM((GSUB, D), F32),
            pltpu.VMEM((GSUB, D), F32),
            pltpu.VMEM_SHARED((ACC, D), F32),
            pltpu.SemaphoreType.DMA,
            pltpu.SemaphoreType.DMA,
            pltpu.SemaphoreType.DMA,
            pltpu.SemaphoreType.DMA,
        ],
    )
    def hop(tab, head, tail, et, rel, zerosD, out,
            hblk, tblk, eblk, ch, rel_l, rows0, rows1, accum,
            gsem0, gsem1, ssem0, ssem1):
        cid = lax.axis_index("c")
        sid = lax.axis_index("s")
        iota = lax.iota(I32, L)
        pltpu.sync_copy(rel, rel_l)

        for q in range(NCHUNK // NC):
            lo = (q * NC + cid) * CQ

            def zb(z, _):
                r0 = (sid * ZPT + z) * RZ
                pltpu.sync_copy(zerosD, accum.at[pl.ds(r0, RZ)])
                return 0
            lax.fori_loop(0, ZPT, zb, 0)
            plsc.subcore_barrier()

            def block(b, _):
                b0 = sid * EPT + b * EBLK
                pltpu.sync_copy(head.at[pl.ds(b0, EBLK)], hblk)
                pltpu.sync_copy(tail.at[pl.ds(b0, EBLK)], tblk)
                pltpu.sync_copy(et.at[pl.ds(b0, EBLK)], eblk)

                def cb(k, _):
                    h = hblk[pl.ds(k * L, L)]
                    in_r = (h >= lo) & (h < lo + CQ)
                    ch[pl.ds(k * L, L)] = jnp.where(in_r, h - lo, DUMP + (h & 63))
                    return 0
                lax.fori_loop(0, EBLK // L, cb, 0)

                def mult(rows, off):
                    def mg(g, _):
                        eo = off + g * L
                        etv = eblk[pl.ds(eo, L)]
                        for l in range(L):
                            ei = etv[l]
                            ri = g * L + l
                            for j in range(D // L):
                                rv = rows[ri, pl.ds(j * L, L)]
                                relv = rel_l[ei, pl.ds(j * L, L)]
                                rows[ri, pl.ds(j * L, L)] = rv * relv
                        return 0
                    lax.fori_loop(0, GSUB // L, mg, 0)

                pltpu.async_copy(tab.at[tblk.at[pl.ds(0, GSUB)]],
                                 rows0, gsem0)

                def sb(p, _):
                    s0 = 2 * p
                    s1 = 2 * p + 1
                    pltpu.make_async_copy(tab.at[tblk.at[pl.ds(0, GSUB)]],
                                          rows0, gsem0).wait()
                    pltpu.async_copy(tab.at[tblk.at[pl.ds(s1 * GSUB, GSUB)]],
                                     rows1, gsem1)
                    mult(rows0, s0 * GSUB)
                    sidx0 = ch.at[pl.ds(s0 * GSUB, GSUB)]
                    pltpu.async_copy(rows0, accum.at[sidx0], ssem0, add=True)
                    pltpu.make_async_copy(rows0, accum.at[sidx0],
                                          ssem0).wait()

                    @pl.when(p + 1 < NSUB // 2)
                    def _():
                        pltpu.async_copy(
                            tab.at[tblk.at[pl.ds((s0 + 2) * GSUB, GSUB)]],
                            rows0, gsem0)

                    pltpu.make_async_copy(tab.at[tblk.at[pl.ds(0, GSUB)]],
                                          rows1, gsem1).wait()
                    mult(rows1, s1 * GSUB)
                    sidx1 = ch.at[pl.ds(s1 * GSUB, GSUB)]
                    pltpu.async_copy(rows1, accum.at[sidx1], ssem1, add=True)
                    pltpu.make_async_copy(rows1, accum.at[sidx1],
                                          ssem1).wait()
                    return 0
                lax.fori_loop(0, NSUB // 2, sb, 0)
                return 0
            lax.fori_loop(0, NBLK, block, 0)
            plsc.subcore_barrier()

            def wb(i, _):
                r0 = (sid + i * NS) * CWB
                pltpu.sync_copy(accum.at[pl.ds(r0, CWB)],
                                out.at[pl.ds(lo + r0, CWB)])
                return 0
            lax.fori_loop(0, (NCH - sid + NS - 1) // NS, wb, 0)
            plsc.subcore_barrier()

    return hop


def _make_gather():
    RPW = B // (NC * NS)
    mesh = plsc.VectorSubcoreMesh(core_axis_name="c", subcore_axis_name="s")

    @functools.partial(
        pl.kernel, mesh=mesh, compiler_params=_SC_PARAMS,
        out_type=jax.ShapeDtypeStruct((3, B, D), F32),
        scratch_types=[
            pltpu.VMEM((RPW,), I32),
            pltpu.VMEM((RPW,), I32),
            pltpu.VMEM((RPW,), I32),
            pltpu.VMEM((RPW, D), F32),
        ],
    )
    def gather(allT, fuT, uids, iids, out, uv, ulv, iv, rows):
        cid = lax.axis_index("c")
        sid = lax.axis_index("s")
        wid = sid * NC + cid
        base = wid * RPW
        pltpu.sync_copy(uids.at[pl.ds(base, RPW)], uv)
        pltpu.sync_copy(iids.at[pl.ds(base, RPW)], iv)

        def adj(k, _):
            ulv[pl.ds(k * L, L)] = uv[pl.ds(k * L, L)] - N_ENTITIES
            return 0
        lax.fori_loop(0, RPW // L, adj, 0)

        for j, (tab, idx) in enumerate(((fuT, ulv), (allT, uv), (allT, iv))):
            pltpu.sync_copy(tab.at[idx], rows)
            pltpu.sync_copy(rows, out.at[j].at[pl.ds(base, RPW)])

    return gather


def _mlp(image_features, text_features, W_img1, b_img1, W_img2, b_img2,
         W_txt1, b_txt1, W_txt2, b_txt2):
    n = image_features.shape[0]
    blk = 1000
    grid = n // blk

    def body(xi_ref, xt_ref, wi1, bi1, wi2, bi2, wt1, bt1, wt2, bt2, o_ref):
        dn = (((1,), (1,)), ((), ()))
        hi = lax.dot_general(xi_ref[...], wi1[...], dn,
                             preferred_element_type=F32) + bi1[...]
        hi = jnp.where(hi > 0, hi, 0.01 * hi)
        oi = lax.dot_general(hi, wi2[...], dn,
                             preferred_element_type=F32) + bi2[...]
        ht = lax.dot_general(xt_ref[...], wt1[...], dn,
                             preferred_element_type=F32) + bt1[...]
        ht = jnp.where(ht > 0, ht, 0.01 * ht)
        ot = lax.dot_general(ht, wt2[...], dn,
                             preferred_element_type=F32) + bt2[...]
        o_ref[...] = jnp.concatenate([oi, ot], axis=1)

    full = lambda a: pl.BlockSpec(a.shape, lambda i: (0,) * a.ndim)
    return pl.pallas_call(
        body,
        grid=(grid,),
        in_specs=[
            pl.BlockSpec((blk, image_features.shape[1]), lambda i: (i, 0)),
            pl.BlockSpec((blk, text_features.shape[1]), lambda i: (i, 0)),
            full(W_img1), full(b_img1), full(W_img2), full(b_img2),
            full(W_txt1), full(b_txt1), full(W_txt2), full(b_txt2),
        ],
        out_specs=pl.BlockSpec((blk, D), lambda i: (i, 0)),
        out_shape=jax.ShapeDtypeStruct((n, D), F32),
    )(image_features, text_features, W_img1, b_img1, W_img2, b_img2,
      W_txt1, b_txt1, W_txt2, b_txt2)


def _scale(raw, counts):
    n = raw.shape[0]
    blk = 1000
    grid = n // blk

    def body(r, c, o):
        inv = 1.0 / jnp.maximum(c[:, :1], 1.0)
        o[...] = r[...] * inv

    bs = pl.BlockSpec((blk, D), lambda i: (i, 0))
    cs = pl.BlockSpec((blk, 16), lambda i: (i, 0))
    return pl.pallas_call(
        body, grid=(grid,), in_specs=[bs, cs], out_specs=bs,
        out_shape=jax.ShapeDtypeStruct((n, D), F32),
    )(raw, counts)


def _scale_combine(raw, counts, ego, agg1):
    n = raw.shape[0]
    blk = 1000
    grid = n // blk

    def body(r, c, e, a, o):
        inv = 1.0 / jnp.maximum(c[:, :1], 1.0)
        o[...] = (e[...] + a[...] + r[...] * inv) * (1.0 / 3.0)

    bs = pl.BlockSpec((blk, D), lambda i: (i, 0))
    cs = pl.BlockSpec((blk, 16), lambda i: (i, 0))
    return pl.pallas_call(
        body, grid=(grid,), in_specs=[bs, cs, bs, bs], out_specs=bs,
        out_shape=jax.ShapeDtypeStruct((n, D), F32),
    )(raw, counts, ego, agg1)


def _gating(gath, g1W, g1b, g2W, g2b, g3W, g3b, g4W, g4b):
    blk = 512
    grid = B // blk

    def body(g, w1, c1, w2, c2, w3, c3, w4, c4, o):
        dn = (((1,), (1,)), ((), ()))
        a = g[0]
        bb = g[1]
        it = g[2]
        a_i, a_t = a[:, :64], a[:, 64:]
        b_i, b_t = bb[:, :64], bb[:, 64:]
        z1 = (lax.dot_general(a_i, w1[...], dn, preferred_element_type=F32)
              + c1[...]
              + lax.dot_general(b_i, w2[...], dn, preferred_element_type=F32)
              + c2[...])
        gi1 = 1.0 / (1.0 + jnp.exp(-z1))
        uf_i = gi1 * a_i + (1.0 - gi1) * b_i
        z2 = (lax.dot_general(a_t, w3[...], dn, preferred_element_type=F32)
              + c3[...]
              + lax.dot_general(b_t, w4[...], dn, preferred_element_type=F32)
              + c4[...])
        gi2 = 1.0 / (1.0 + jnp.exp(-z2))
        uf_t = gi2 * a_t + (1.0 - gi2) * b_t
        uf = jnp.concatenate([uf_i, uf_t], axis=1)
        s = jnp.sum(uf * it, axis=1)
        o[0, :] = 1.0 / (1.0 + jnp.exp(-s))

    full = lambda a: pl.BlockSpec(a.shape, lambda i: (0,) * a.ndim)
    out = pl.pallas_call(
        body, grid=(grid,),
        in_specs=[pl.BlockSpec((3, blk, D), lambda i: (0, i, 0)),
                  full(g1W), full(g1b), full(g2W), full(g2b),
                  full(g3W), full(g3b), full(g4W), full(g4b)],
        out_specs=pl.BlockSpec((1, blk), lambda i: (0, i)),
        out_shape=jax.ShapeDtypeStruct((1, B), F32),
    )(gath, g1W, g1b, g2W, g2b, g3W, g3b, g4W, g4b)
    return out.reshape(B)


_deg_kg = _make_deg(N_NODES, KG_EPAD)
_deg_ukg = _make_deg(N_USERS, UKG_EPAD)
_hop_kg = _make_hop(N_NODES, KG_EPAD, 32, 4)
_hop_ukg = _make_hop(N_USERS, UKG_EPAD, 8, 2)
_gather3 = _make_gather()


def _pad_edges(head, tail, et, epad, n):
    e = head.shape[0]
    pad = epad - e
    headp = jnp.concatenate([head, jnp.full((pad,), n, I32)])
    tailp = jnp.concatenate([tail, jnp.zeros((pad,), I32)])
    etp = jnp.concatenate([et, jnp.zeros((pad,), I32)])
    return headp, tailp, etp


_STAGE = 15


def _seg_raw(tab, head, tail, et, rel, n):
    return jax.ops.segment_sum(tab[tail] * rel[et], head, num_segments=n)


def kernel(user_ids, item_ids, edge_index, edge_type, ukg_edge_index,
           ukg_edge_type, image_features, text_features,
           W_img1, b_img1, W_img2, b_img2, W_txt1, b_txt1, W_txt2, b_txt2,
           other_emb_image, other_emb_text, rel_emb_image, rel_emb_text,
           ukg_rel_emb_image, ukg_rel_emb_text,
           g1W, g1b, g2W, g2b, g3W, g3b, g4W, g4b):
    zerosD = jnp.zeros((RZ, D), F32)
    zeros16 = jnp.zeros((RZ, 16), F32)
    ones16 = jnp.ones((GSUB, 16), F32)

    mlp_out = _mlp(image_features, text_features, W_img1, b_img1, W_img2,
                   b_img2, W_txt1, b_txt1, W_txt2, b_txt2)
    other = jnp.concatenate([other_emb_image, other_emb_text], axis=1)
    ego = jnp.concatenate([mlp_out, other], axis=0)
    rel = jnp.concatenate([rel_emb_image, rel_emb_text], axis=1)
    urel = jnp.concatenate([ukg_rel_emb_image, ukg_rel_emb_text], axis=1)

    head, tail = edge_index[0], edge_index[1]
    uhead, utail = ukg_edge_index[0], ukg_edge_index[1]
    headp, tailp, etp = _pad_edges(head, tail, edge_type, KG_EPAD, N_NODES)
    uheadp, utailp, uetp = _pad_edges(uhead, utail, ukg_edge_type,
                                      UKG_EPAD, N_USERS)

    if _STAGE & 1:
        cpad = _deg_kg(headp)
        hp = _round_up(N_NODES // 2, 128)
        counts1 = jnp.concatenate([cpad[:N_NODES // 2],
                                   cpad[hp:hp + N_NODES // 2]])
        counts = jnp.broadcast_to(counts1[:, None], (N_NODES, 16))
        ucpad = _deg_ukg(uheadp)
        uhp = _round_up(N_USERS // 2, 128)
        ucounts1 = jnp.concatenate([ucpad[:N_USERS // 2],
                                    ucpad[uhp:uhp + N_USERS // 2]])
        ucounts = jnp.broadcast_to(ucounts1[:, None], (N_USERS, 16))
    else:
        counts = jnp.broadcast_to(jax.ops.segment_sum(
            jnp.ones(head.shape[0], F32), head,
            num_segments=N_NODES)[:, None], (N_NODES, 16))
        ucounts = jnp.broadcast_to(jax.ops.segment_sum(
            jnp.ones(uhead.shape[0], F32), uhead,
            num_segments=N_USERS)[:, None], (N_USERS, 16))

    if _STAGE & 2:
        raw1 = _hop_kg(ego, headp, tailp, etp, rel, zerosD)
        agg1 = _scale(raw1, counts)
        raw2 = _hop_kg(agg1, headp, tailp, etp, rel, zerosD)
    else:
        raw1 = _seg_raw(ego, head, tail, edge_type, rel, N_NODES)
        agg1 = _scale(raw1, counts)
        raw2 = _seg_raw(agg1, head, tail, edge_type, rel, N_NODES)
    allT = _scale_combine(raw2, counts, ego, agg1)

    uego = allT[N_ENTITIES:]
    if _STAGE & 4:
        uraw1 = _hop_ukg(uego, uheadp, utailp, uetp, urel, zerosD)
        uagg1 = _scale(uraw1, ucounts)
        uraw2 = _hop_ukg(uagg1, uheadp, utailp, uetp, urel, zerosD)
    else:
        uraw1 = _seg_raw(uego, uhead, utail, ukg_edge_type, urel, N_USERS)
        uagg1 = _scale(uraw1, ucounts)
        uraw2 = _seg_raw(uagg1, uhead, utail, ukg_edge_type, urel, N_USERS)
    fuT = _scale_combine(uraw2, ucounts, uego, uagg1)

    if _STAGE & 8:
        gath = _gather3(allT, fuT, user_ids, item_ids)
    else:
        ul = user_ids - N_ENTITIES
        gath = jnp.stack([fuT[ul], allT[user_ids], allT[item_ids]])
    return _gating(gath, g1W, g1b, g2W, g2b, g3W, g3b, g4W, g4b)
